# baseline (speedup 1.0000x reference)
import functools

import jax
import jax.numpy as jnp
from jax import lax
from jax.experimental import pallas as pl
from jax.experimental.pallas import tpu as pltpu
from jax.experimental.pallas import tpu_sc as plsc

_CHANNELS = 64
_N_TOKENS = 16384
_LANES = 16


def _build_sc_kernel(B, D, C, NC, NS):
    NW = NC * NS
    per_w = B // NW
    n_chunks = per_w // C
    mesh = plsc.VectorSubcoreMesh(core_axis_name="c", subcore_axis_name="s")

    @functools.partial(
        pl.kernel,
        mesh=mesh,
        out_type=jax.ShapeDtypeStruct((3, B // 2, 2 * D), jnp.float32),
        scratch_types=[
            pltpu.VMEM((C,), jnp.int32),
            pltpu.VMEM((C,), jnp.int32),
            pltpu.VMEM((C,), jnp.int32),
            pltpu.VMEM((C,), jnp.int32),
            pltpu.VMEM((C,), jnp.int32),
            pltpu.VMEM((C, 2 * D), jnp.float32),
            pltpu.VMEM((C, 2 * D), jnp.float32),
            pltpu.VMEM((C, 2 * D), jnp.float32),
            pltpu.VMEM((C, 2 * D), jnp.float32),
            pltpu.VMEM((C // 2, 2 * D), jnp.float32),
            pltpu.VMEM((C // 2, 2 * D), jnp.float32),
            pltpu.VMEM((C // 2, 2 * D), jnp.float32),
            pltpu.SemaphoreType.DMA,
        ],
    )
    def k(xa, xm, xr, xt, wa2, wm2, wr2, wt2, out,
          ia, iq, im, ir, it, A, M, R, T, As, Ms, Rs, sem):
        wid = lax.axis_index("s") * NC + lax.axis_index("c")
        base0 = wid * per_w

        def chunk(ci, _):
            base = pl.multiple_of(base0 + ci * C, C)
            pltpu.sync_copy(xa.at[pl.ds(base, C)], ia)
            pltpu.sync_copy(xm.at[pl.ds(base, C)], im)
            pltpu.sync_copy(xr.at[pl.ds(base, C)], ir)
            pltpu.sync_copy(xt.at[pl.ds(base, C)], it)

            def halve(g, _2):
                sl = pl.ds(g * _LANES, _LANES)
                iq[sl] = ia[sl] >> 1
                return 0

            lax.fori_loop(0, C // _LANES, halve, 0)
            cps = [
                pltpu.async_copy(wa2.at[iq], A, sem),
                pltpu.async_copy(wm2.at[im], M, sem),
                pltpu.async_copy(wr2.at[ir], R, sem),
                pltpu.async_copy(wt2.at[it], T, sem),
            ]
            for cp in cps:
                cp.wait()

            def row(g, _2):
                xv = ia[pl.ds(g * _LANES, _LANES)]
                for l in range(_LANES):
                    i = g * _LANES + l
                    srow = g * (_LANES // 2) + (l // 2)
                    soff = D * (l & 1)
                    col0 = (xv[l] & 1) * D
                    for j in range(D // _LANES):
                        tv = T[i, pl.ds(j * _LANES, _LANES)]
                        av = A[i, pl.ds(col0 + j * _LANES, _LANES)]
                        mv = M[i, pl.ds(j * _LANES, _LANES)]
                        rv = R[i, pl.ds(j * _LANES, _LANES)]
                        dsl = pl.ds(soff + j * _LANES, _LANES)
                        As[srow, dsl] = av + tv
                        Ms[srow, dsl] = mv + tv
                        Rs[srow, dsl] = rv + tv
                return 0

            lax.fori_loop(0, C // _LANES, row, 0)
            hbase = pl.multiple_of(base // 2, C // 2)
            pltpu.sync_copy(As, out.at[0, pl.ds(hbase, C // 2)])
            pltpu.sync_copy(Ms, out.at[1, pl.ds(hbase, C // 2)])
            pltpu.sync_copy(Rs, out.at[2, pl.ds(hbase, C // 2)])
            return 0

        lax.fori_loop(0, n_chunks, chunk, 0)

    return k


def kernel(x_action, x_mode, x_readout, t, W_action, W_mode, W_readout, W_time):
    info = plsc.get_sparse_core_info()
    k = _build_sc_kernel(_N_TOKENS, _CHANNELS, 128, info.num_cores,
                         info.num_subcores)
    wa2 = W_action.reshape(500000, 2 * _CHANNELS)
    wm2 = jnp.concatenate([W_mode, W_mode], axis=1)
    wr2 = jnp.concatenate([W_readout, W_readout], axis=1)
    wt2 = jnp.concatenate([W_time, W_time], axis=1)
    out128 = k(x_action.astype(jnp.int32), x_mode.astype(jnp.int32),
               x_readout.astype(jnp.int32), t.astype(jnp.int32),
               wa2, wm2, wr2, wt2)
    return out128.reshape(3, _N_TOKENS, _CHANNELS)


# speedup vs baseline: 2.7261x; 2.7261x over previous
import functools

import jax
import jax.numpy as jnp
from jax import lax
from jax.experimental import pallas as pl
from jax.experimental.pallas import tpu as pltpu
from jax.experimental.pallas import tpu_sc as plsc

_CHANNELS = 64
_N_TOKENS = 16384
_LANES = 16
_MB = 4
_NBUF = 2


def _build_sc_kernel(B, D, C, NC, NS):
    NW = NC * NS
    per_w = B // NW
    n_chunks = per_w // C
    n_mb = C // _MB
    mesh = plsc.VectorSubcoreMesh(core_axis_name="c", subcore_axis_name="s")

    @functools.partial(
        pl.kernel,
        mesh=mesh,
        out_type=jax.ShapeDtypeStruct((3, B // 2, 2 * D), jnp.float32),
        compiler_params=pltpu.CompilerParams(needs_layout_passes=False),
        scratch_types=[
            pltpu.VMEM((C,), jnp.int32),
            pltpu.VMEM((C,), jnp.int32),
            pltpu.VMEM((C,), jnp.int32),
            pltpu.VMEM((C,), jnp.int32),
            pltpu.VMEM((_NBUF, _MB, D, 128), jnp.float32),
            pltpu.VMEM((C, 2 * D), jnp.float32),
            pltpu.VMEM((C, 2 * D), jnp.float32),
            pltpu.VMEM((4, 2 * D), jnp.float32),
            pltpu.VMEM((C // 2, 2 * D), jnp.float32),
            pltpu.VMEM((C // 2, 2 * D), jnp.float32),
            pltpu.VMEM((C // 2, 2 * D), jnp.float32),
            pltpu.SemaphoreType.DMA,
            pltpu.SemaphoreType.DMA,
            pltpu.SemaphoreType.DMA,
        ],
    )
    def k(xa, xm, xr, xt, waT, wm2, wr2, wt2, out,
          ia, im, ir, it, W, M, T, Rt, As, Ms, Rs, sem, ws0, ws1):
        wid = lax.axis_index("s") * NC + lax.axis_index("c")
        base0 = wid * per_w
        wsems = [ws0, ws1]
        rows16 = [jnp.arange(_LANES, dtype=jnp.int32) + j * _LANES
                  for j in range(D // _LANES)]

        pltpu.sync_copy(wr2, Rt)

        def fire_mb(mb_dyn, buf):
            xv = ia[pl.ds(mb_dyn * _MB, _LANES)]
            for q in range(_MB):
                s = pl.multiple_of((xv[q] >> 7) * 128, 128)
                pltpu.async_copy(
                    waT.at[:, pl.ds(s, 128)], W.at[buf, q], wsems[buf]
                )

        def drain_mb(buf):
            for _ in range(_MB):
                pltpu.make_async_copy(
                    waT.at[:, pl.ds(0, 128)], W.at[buf, 0], wsems[buf]
                ).wait()

        def proc_mb(mb_dyn, buf):
            xv = ia[pl.ds(mb_dyn * _MB, _LANES)]
            for q in range(_MB):
                i = mb_dyn * _MB + q
                col = xv[q] & 127
                colv = jnp.full((_LANES,), col, dtype=jnp.int32)
                bufv = jnp.full((_LANES,), q, dtype=jnp.int32)
                srow = mb_dyn * (_MB // 2) + (q // 2)
                soff = D * (q & 1)
                xrs = ir[pl.ds(mb_dyn * _MB, _LANES)]
                rrow = xrs[q]
                for j in range(D // _LANES):
                    av = plsc.load_gather(
                        W.at[buf], [bufv, rows16[j], colv])
                    tv = T[i, pl.ds(j * _LANES, _LANES)]
                    mv = M[i, pl.ds(j * _LANES, _LANES)]
                    rv = Rt[rrow, pl.ds(j * _LANES, _LANES)]
                    dsl = pl.ds(soff + j * _LANES, _LANES)
                    As[srow, dsl] = av + tv
                    Ms[srow, dsl] = mv + tv
                    Rs[srow, dsl] = rv + tv

        def chunk(ci, _):
            base = pl.multiple_of(base0 + ci * C, C)
            pltpu.sync_copy(xa.at[pl.ds(base, C)], ia)
            pltpu.sync_copy(xm.at[pl.ds(base, C)], im)
            pltpu.sync_copy(xr.at[pl.ds(base, C)], ir)
            pltpu.sync_copy(xt.at[pl.ds(base, C)], it)
            cps = [
                pltpu.async_copy(wm2.at[im], M, sem),
                pltpu.async_copy(wt2.at[it], T, sem),
            ]
            fire_mb(0, 0)
            fire_mb(1, 1)
            for cp in cps:
                cp.wait()

            def body(u, _2):
                mb0 = u * 2
                drain_mb(0)
                proc_mb(mb0, 0)

                @pl.when(mb0 + 2 < n_mb)
                def _f0():
                    fire_mb(mb0 + 2, 0)

                drain_mb(1)
                proc_mb(mb0 + 1, 1)

                @pl.when(mb0 + 3 < n_mb)
                def _f1():
                    fire_mb(mb0 + 3, 1)

                return 0

            lax.fori_loop(0, n_mb // 2, body, 0)
            hbase = pl.multiple_of(base // 2, C // 2)
            pltpu.sync_copy(As, out.at[0, pl.ds(hbase, C // 2)])
            pltpu.sync_copy(Ms, out.at[1, pl.ds(hbase, C // 2)])
            pltpu.sync_copy(Rs, out.at[2, pl.ds(hbase, C // 2)])
            return 0

        lax.fori_loop(0, n_chunks, chunk, 0)

    return k


def kernel(x_action, x_mode, x_readout, t, W_action, W_mode, W_readout, W_time):
    info = plsc.get_sparse_core_info()
    k = _build_sc_kernel(_N_TOKENS, _CHANNELS, 128, info.num_cores,
                         info.num_subcores)
    wm2 = jnp.concatenate([W_mode, W_mode], axis=1)
    wr2 = jnp.concatenate([W_readout, W_readout], axis=1)
    wt2 = jnp.concatenate([W_time, W_time], axis=1)
    out128 = k(x_action.astype(jnp.int32), x_mode.astype(jnp.int32),
               x_readout.astype(jnp.int32), t.astype(jnp.int32),
               W_action.T, wm2, wr2, wt2)
    return out128.reshape(3, _N_TOKENS, _CHANNELS)
